# baseline (speedup 1.0000x reference)
import functools

import jax
import jax.numpy as jnp
from jax.experimental import pallas as pl
from jax.experimental.pallas import tpu as pltpu

_EPS = 1e-5
_NHEAD = 8

_TB = (((1,), (1,)), ((), ()))
_QK = (((2,), (2,)), ((0,), (0,)))
_PV = (((2,), (1,)), ((0,), (0,)))


def _dot(a, b, dims):
    return jax.lax.dot_general(a, b, dims, preferred_element_type=jnp.float32)


def _ln(y, g, b):
    mean = jnp.mean(y, axis=-1, keepdims=True)
    msq = jnp.mean(y * y, axis=-1, keepdims=True)
    var = msq - mean * mean
    return (y - mean) * jax.lax.rsqrt(var + _EPS) * g + b


def _heads(t, s, bt):
    d = t.shape[-1]
    hd = d // _NHEAD
    return (t.reshape(s, bt, _NHEAD, hd)
             .transpose(1, 2, 0, 3)
             .reshape(bt * _NHEAD, s, hd))


def _unheads(t, s, bt):
    n, _, hd = t.shape
    return (t.reshape(bt, _NHEAD, s, hd)
             .transpose(2, 0, 1, 3)
             .reshape(s * bt, _NHEAD * hd))


def _softmax(scores):
    e = jnp.exp(scores)
    return e * (1.0 / jnp.sum(e, axis=-1, keepdims=True))


def _self_block(x2, s, bt, scale, wq, bq, wk, bk, wv, bv, wo, bo, g, b):
    q = (_dot(x2, wq, _TB) + bq) * scale
    k = _dot(x2, wk, _TB) + bk
    v = _dot(x2, wv, _TB) + bv
    probs = _softmax(_dot(_heads(q, s, bt), _heads(k, s, bt), _QK))
    ctx = _unheads(_dot(probs, _heads(v, s, bt), _PV), s, bt)
    att = _dot(ctx, wo, _TB) + bo
    return _ln(x2 + att, g, b)


def _cross_block(y2, ykv, s, bt, scale,
                 wq, bq, wk, bk, wv, bv, wo, bo, g1, b1):
    q = (_dot(y2, wq, _TB) + bq) * scale
    k = _dot(ykv, wk, _TB) + bk
    v = _dot(ykv, wv, _TB) + bv
    probs = _softmax(_dot(_heads(q, s, bt), _heads(k, s, bt), _QK))
    ctx = _unheads(_dot(probs, _heads(v, s, bt), _PV), s, bt)
    att = _dot(ctx, wo, _TB) + bo
    y = _ln(y2 + att, g1, b1)
    attn = jnp.mean(probs.reshape(bt, _NHEAD, s, s), axis=1)
    return y, attn


def _ffn_block(y, w1, bf1, w2, bf2, g2, b2):
    h = jnp.maximum(_dot(y, w1, _TB) + bf1, 0.0)
    f = _dot(h, w2, _TB) + bf2
    return _ln(y + f, g2, b2)


def _stage1_kernel(s1_ref, s2_ref,
                   a1wq, a1bq, a1wk, a1bk, a1wv, a1bv, a1wo, a1bo, n11g, n11b,
                   a2wq_hbm, a2bq, a2wk_hbm, a2bk, a2wv_hbm, a2bv,
                   a2wo_hbm, a2bo, n21g, n21b,
                   y1_ref, y2_ref, b2q_v, b2k_v, b2v_v, b2o_v, b2_sem,
                   *, scale):
    s, bt, d = s1_ref.shape
    j = pl.program_id(0)

    @pl.when(j == 0)
    def _():
        pltpu.make_async_copy(a2wq_hbm, b2q_v, b2_sem.at[0]).start()
        pltpu.make_async_copy(a2wk_hbm, b2k_v, b2_sem.at[1]).start()
        pltpu.make_async_copy(a2wv_hbm, b2v_v, b2_sem.at[2]).start()
        pltpu.make_async_copy(a2wo_hbm, b2o_v, b2_sem.at[3]).start()

    x1 = s1_ref[...].reshape(s * bt, d)
    x2 = s2_ref[...].reshape(s * bt, d)
    y1_ref[...] = _self_block(
        x1, s, bt, scale, a1wq[...], a1bq[...], a1wk[...], a1bk[...],
        a1wv[...], a1bv[...], a1wo[...], a1bo[...], n11g[...], n11b[...]
    ).reshape(s, bt, d)

    @pl.when(j == 0)
    def _():
        pltpu.make_async_copy(a2wq_hbm, b2q_v, b2_sem.at[0]).wait()
        pltpu.make_async_copy(a2wk_hbm, b2k_v, b2_sem.at[1]).wait()
        pltpu.make_async_copy(a2wv_hbm, b2v_v, b2_sem.at[2]).wait()
        pltpu.make_async_copy(a2wo_hbm, b2o_v, b2_sem.at[3]).wait()

    y2_ref[...] = _self_block(
        x2, s, bt, scale, b2q_v[...], a2bq[...], b2k_v[...], a2bk[...],
        b2v_v[...], a2bv[...], b2o_v[...], a2bo[...], n21g[...], n21b[...]
    ).reshape(s, bt, d)


def _stage2_kernel(y1_ref, y2_ref,
                   c1wq, c1bq, c1wk, c1bk, c1wv, c1bv, c1wo, c1bo, n12g, n12b,
                   l11w_hbm, l11b, l12w_hbm, l12b, n13g, n13b,
                   c2wq_hbm, c2bq, c2wk_hbm, c2bk, c2wv_hbm, c2bv,
                   c2wo_hbm, c2bo, n22g, n22b,
                   l21w_hbm, l21b, l22w_hbm, l22b, n23g, n23b,
                   out1_ref, out2_ref, attn1_ref, attn2_ref,
                   w11_v, w12_v, w21_v, w22_v,
                   b2q_v, b2k_v, b2v_v, b2o_v, late_sem, *, scale):
    s, bt, d = y1_ref.shape
    j = pl.program_id(0)

    @pl.when(j == 0)
    def _():
        pltpu.make_async_copy(c2wq_hbm, b2q_v, late_sem.at[4]).start()
        pltpu.make_async_copy(c2wk_hbm, b2k_v, late_sem.at[5]).start()
        pltpu.make_async_copy(c2wv_hbm, b2v_v, late_sem.at[6]).start()
        pltpu.make_async_copy(c2wo_hbm, b2o_v, late_sem.at[7]).start()
        pltpu.make_async_copy(l11w_hbm, w11_v, late_sem.at[0]).start()
        pltpu.make_async_copy(l12w_hbm, w12_v, late_sem.at[1]).start()
        pltpu.make_async_copy(l21w_hbm, w21_v, late_sem.at[2]).start()
        pltpu.make_async_copy(l22w_hbm, w22_v, late_sem.at[3]).start()

    y1 = y1_ref[...].reshape(s * bt, d)
    y2 = y2_ref[...].reshape(s * bt, d)

    m1, at1 = _cross_block(
        y1, y2, s, bt, scale,
        c1wq[...], c1bq[...], c1wk[...], c1bk[...], c1wv[...], c1bv[...],
        c1wo[...], c1bo[...], n12g[...], n12b[...])

    @pl.when(j == 0)
    def _():
        pltpu.make_async_copy(c2wq_hbm, b2q_v, late_sem.at[4]).wait()
        pltpu.make_async_copy(c2wk_hbm, b2k_v, late_sem.at[5]).wait()
        pltpu.make_async_copy(c2wv_hbm, b2v_v, late_sem.at[6]).wait()
        pltpu.make_async_copy(c2wo_hbm, b2o_v, late_sem.at[7]).wait()

    m2, at2 = _cross_block(
        y2, y1, s, bt, scale,
        b2q_v[...], c2bq[...], b2k_v[...], c2bk[...], b2v_v[...], c2bv[...],
        b2o_v[...], c2bo[...], n22g[...], n22b[...])

    @pl.when(j == 0)
    def _():
        pltpu.make_async_copy(l11w_hbm, w11_v, late_sem.at[0]).wait()
        pltpu.make_async_copy(l12w_hbm, w12_v, late_sem.at[1]).wait()
        pltpu.make_async_copy(l21w_hbm, w21_v, late_sem.at[2]).wait()
        pltpu.make_async_copy(l22w_hbm, w22_v, late_sem.at[3]).wait()

    o1 = _ffn_block(m1, w11_v[...], l11b[...], w12_v[...], l12b[...],
                    n13g[...], n13b[...])
    o2 = _ffn_block(m2, w21_v[...], l21b[...], w22_v[...], l22b[...],
                    n23g[...], n23b[...])
    out1_ref[...] = o1.reshape(s, bt, d)
    out2_ref[...] = o2.reshape(s, bt, d)
    attn1_ref[...] = at1
    attn2_ref[...] = at2


def _vec(b):
    return b.reshape(1, b.shape[0])


def kernel(src1, src2, self_attn1_wq, self_attn1_wk, self_attn1_wv, self_attn1_wo, self_attn1_bq, self_attn1_bk, self_attn1_bv, self_attn1_bo, self_attn2_wq, self_attn2_wk, self_attn2_wv, self_attn2_wo, self_attn2_bq, self_attn2_bk, self_attn2_bv, self_attn2_bo, multihead_attn1_wq, multihead_attn1_wk, multihead_attn1_wv, multihead_attn1_wo, multihead_attn1_bq, multihead_attn1_bk, multihead_attn1_bv, multihead_attn1_bo, multihead_attn2_wq, multihead_attn2_wk, multihead_attn2_wv, multihead_attn2_wo, multihead_attn2_bq, multihead_attn2_bk, multihead_attn2_bv, multihead_attn2_bo, lin11_w, lin11_b, lin12_w, lin12_b, lin21_w, lin21_b, lin22_w, lin22_b, norm11_g, norm11_b, norm12_g, norm12_b, norm13_g, norm13_b, norm21_g, norm21_b, norm22_g, norm22_b, norm23_g, norm23_b):
    s, b, d = src1.shape
    hd = d // _NHEAD
    scale = 1.0 / (hd ** 0.5)
    bt = 8 if b % 8 == 0 else b
    nj = b // bt

    xspec = pl.BlockSpec((s, bt, d), lambda j: (0, j, 0))

    def wspec(arr):
        nd = arr.ndim
        return pl.BlockSpec(arr.shape, lambda j, _n=nd: (0,) * _n)

    ops1 = [
        src1, src2,
        self_attn1_wq, _vec(self_attn1_bq), self_attn1_wk, _vec(self_attn1_bk),
        self_attn1_wv, _vec(self_attn1_bv), self_attn1_wo, _vec(self_attn1_bo),
        _vec(norm11_g), _vec(norm11_b),
        self_attn2_wq, _vec(self_attn2_bq), self_attn2_wk, _vec(self_attn2_bk),
        self_attn2_wv, _vec(self_attn2_bv), self_attn2_wo, _vec(self_attn2_bo),
        _vec(norm21_g), _vec(norm21_b),
    ]
    b2_idx1 = {12, 14, 16, 18}
    in_specs1 = []
    for i, a in enumerate(ops1):
        if i < 2:
            in_specs1.append(xspec)
        elif i in b2_idx1:
            in_specs1.append(pl.BlockSpec(memory_space=pl.ANY))
        else:
            in_specs1.append(wspec(a))
    y1, y2 = pl.pallas_call(
        functools.partial(_stage1_kernel, scale=scale),
        out_shape=(jax.ShapeDtypeStruct((s, b, d), jnp.float32),
                   jax.ShapeDtypeStruct((s, b, d), jnp.float32)),
        grid=(nj,),
        in_specs=in_specs1,
        out_specs=(xspec, xspec),
        scratch_shapes=[pltpu.VMEM((d, d), jnp.float32),
                        pltpu.VMEM((d, d), jnp.float32),
                        pltpu.VMEM((d, d), jnp.float32),
                        pltpu.VMEM((d, d), jnp.float32),
                        pltpu.SemaphoreType.DMA((4,))],
        compiler_params=pltpu.CompilerParams(
            dimension_semantics=("arbitrary",)),
    )(*ops1)

    f = lin11_w.shape[0]
    ops2 = [
        y1, y2,
        multihead_attn1_wq, _vec(multihead_attn1_bq),
        multihead_attn1_wk, _vec(multihead_attn1_bk),
        multihead_attn1_wv, _vec(multihead_attn1_bv),
        multihead_attn1_wo, _vec(multihead_attn1_bo),
        _vec(norm12_g), _vec(norm12_b),
        lin11_w, _vec(lin11_b), lin12_w, _vec(lin12_b),
        _vec(norm13_g), _vec(norm13_b),
        multihead_attn2_wq, _vec(multihead_attn2_bq),
        multihead_attn2_wk, _vec(multihead_attn2_bk),
        multihead_attn2_wv, _vec(multihead_attn2_bv),
        multihead_attn2_wo, _vec(multihead_attn2_bo),
        _vec(norm22_g), _vec(norm22_b),
        lin21_w, _vec(lin21_b), lin22_w, _vec(lin22_b),
        _vec(norm23_g), _vec(norm23_b),
    ]
    late_idx2 = {12, 14, 28, 30, 18, 20, 22, 24}
    in_specs2 = []
    for i, a in enumerate(ops2):
        if i < 2:
            in_specs2.append(xspec)
        elif i in late_idx2:
            in_specs2.append(pl.BlockSpec(memory_space=pl.ANY))
        else:
            in_specs2.append(wspec(a))
    out1, out2, attn1, attn2 = pl.pallas_call(
        functools.partial(_stage2_kernel, scale=scale),
        out_shape=(jax.ShapeDtypeStruct((s, b, d), jnp.float32),
                   jax.ShapeDtypeStruct((s, b, d), jnp.float32),
                   jax.ShapeDtypeStruct((b, s, s), jnp.float32),
                   jax.ShapeDtypeStruct((b, s, s), jnp.float32)),
        grid=(nj,),
        in_specs=in_specs2,
        out_specs=(xspec, xspec,
                   pl.BlockSpec((bt, s, s), lambda j: (j, 0, 0)),
                   pl.BlockSpec((bt, s, s), lambda j: (j, 0, 0))),
        scratch_shapes=[pltpu.VMEM((f, d), jnp.float32),
                        pltpu.VMEM((d, f), jnp.float32),
                        pltpu.VMEM((f, d), jnp.float32),
                        pltpu.VMEM((d, f), jnp.float32),
                        pltpu.VMEM((d, d), jnp.float32),
                        pltpu.VMEM((d, d), jnp.float32),
                        pltpu.VMEM((d, d), jnp.float32),
                        pltpu.VMEM((d, d), jnp.float32),
                        pltpu.SemaphoreType.DMA((8,))],
        compiler_params=pltpu.CompilerParams(
            dimension_semantics=("arbitrary",)),
    )(*ops2)
    return out1, out2, attn1, attn2


# speedup vs baseline: 1.0262x; 1.0262x over previous
import functools

import jax
import jax.numpy as jnp
from jax.experimental import pallas as pl
from jax.experimental.pallas import tpu as pltpu

_EPS = 1e-5
_NHEAD = 8

_TB = (((1,), (1,)), ((), ()))
_QK = (((2,), (2,)), ((0,), (0,)))
_PV = (((2,), (1,)), ((0,), (0,)))


def _dot(a, b, dims):
    return jax.lax.dot_general(a, b, dims, preferred_element_type=jnp.float32)


def _ln(y, g, b):
    mean = jnp.mean(y, axis=-1, keepdims=True)
    msq = jnp.mean(y * y, axis=-1, keepdims=True)
    var = msq - mean * mean
    return (y - mean) * jax.lax.rsqrt(var + _EPS) * g + b


def _heads(t, s, bt):
    d = t.shape[-1]
    hd = d // _NHEAD
    return (t.reshape(s, bt, _NHEAD, hd)
             .transpose(1, 2, 0, 3)
             .reshape(bt * _NHEAD, s, hd))


def _unheads(t, s, bt):
    n, _, hd = t.shape
    return (t.reshape(bt, _NHEAD, s, hd)
             .transpose(2, 0, 1, 3)
             .reshape(s * bt, _NHEAD * hd))


def _softmax(scores):
    e = jnp.exp(scores)
    return e * (1.0 / jnp.sum(e, axis=-1, keepdims=True))


def _self_block(x2, s, bt, scale, wq, bq, wk, bk, wv, bv, wo, bo, g, b):
    q = (_dot(x2, wq, _TB) + bq) * scale
    k = _dot(x2, wk, _TB) + bk
    v = _dot(x2, wv, _TB) + bv
    probs = _softmax(_dot(_heads(q, s, bt), _heads(k, s, bt), _QK))
    ctx = _unheads(_dot(probs, _heads(v, s, bt), _PV), s, bt)
    att = _dot(ctx, wo, _TB) + bo
    return _ln(x2 + att, g, b)


def _cross_block(y2, ykv, s, bt, scale,
                 wq, bq, wk, bk, wv, bv, wo, bo, g1, b1):
    q = (_dot(y2, wq, _TB) + bq) * scale
    k = _dot(ykv, wk, _TB) + bk
    v = _dot(ykv, wv, _TB) + bv
    probs = _softmax(_dot(_heads(q, s, bt), _heads(k, s, bt), _QK))
    ctx = _unheads(_dot(probs, _heads(v, s, bt), _PV), s, bt)
    att = _dot(ctx, wo, _TB) + bo
    y = _ln(y2 + att, g1, b1)
    attn = jnp.mean(probs.reshape(bt, _NHEAD, s, s), axis=1)
    return y, attn


def _ffn_block(y, w1, bf1, w2, bf2, g2, b2):
    h = jnp.maximum(_dot(y, w1, _TB) + bf1, 0.0)
    f = _dot(h, w2, _TB) + bf2
    return _ln(y + f, g2, b2)


def _stage1_kernel(s1_ref, s2_ref,
                   a1wq, a1bq, a1wk, a1bk, a1wv, a1bv, a1wo, a1bo, n11g, n11b,
                   a2wq, a2bq, a2wk, a2bk, a2wv, a2bv, a2wo, a2bo, n21g, n21b,
                   y1_ref, y2_ref, *, scale):
    s, bt, d = s1_ref.shape
    x1 = s1_ref[...].reshape(s * bt, d)
    x2 = s2_ref[...].reshape(s * bt, d)
    y1_ref[...] = _self_block(
        x1, s, bt, scale, a1wq[...], a1bq[...], a1wk[...], a1bk[...],
        a1wv[...], a1bv[...], a1wo[...], a1bo[...], n11g[...], n11b[...]
    ).reshape(s, bt, d)
    y2_ref[...] = _self_block(
        x2, s, bt, scale, a2wq[...], a2bq[...], a2wk[...], a2bk[...],
        a2wv[...], a2bv[...], a2wo[...], a2bo[...], n21g[...], n21b[...]
    ).reshape(s, bt, d)


def _stage2_kernel(y1_ref, y2_ref,
                   c1wq, c1bq, c1wk, c1bk, c1wv, c1bv, c1wo, c1bo, n12g, n12b,
                   l11w_hbm, l11b, l12w_hbm, l12b, n13g, n13b,
                   c2wq, c2bq, c2wk, c2bk, c2wv, c2bv, c2wo, c2bo, n22g, n22b,
                   l21w_hbm, l21b, l22w_hbm, l22b, n23g, n23b,
                   out1_ref, out2_ref, attn1_ref, attn2_ref,
                   w11_v, w12_v, w21_v, w22_v, ffn_sem, *, scale):
    s, bt, d = y1_ref.shape
    j = pl.program_id(0)

    @pl.when(j == 0)
    def _():
        pltpu.make_async_copy(l11w_hbm, w11_v, ffn_sem.at[0]).start()
        pltpu.make_async_copy(l12w_hbm, w12_v, ffn_sem.at[1]).start()
        pltpu.make_async_copy(l21w_hbm, w21_v, ffn_sem.at[2]).start()
        pltpu.make_async_copy(l22w_hbm, w22_v, ffn_sem.at[3]).start()

    y1 = y1_ref[...].reshape(s * bt, d)
    y2 = y2_ref[...].reshape(s * bt, d)

    m1, at1 = _cross_block(
        y1, y2, s, bt, scale,
        c1wq[...], c1bq[...], c1wk[...], c1bk[...], c1wv[...], c1bv[...],
        c1wo[...], c1bo[...], n12g[...], n12b[...])
    m2, at2 = _cross_block(
        y2, y1, s, bt, scale,
        c2wq[...], c2bq[...], c2wk[...], c2bk[...], c2wv[...], c2bv[...],
        c2wo[...], c2bo[...], n22g[...], n22b[...])

    @pl.when(j == 0)
    def _():
        pltpu.make_async_copy(l11w_hbm, w11_v, ffn_sem.at[0]).wait()
        pltpu.make_async_copy(l12w_hbm, w12_v, ffn_sem.at[1]).wait()
        pltpu.make_async_copy(l21w_hbm, w21_v, ffn_sem.at[2]).wait()
        pltpu.make_async_copy(l22w_hbm, w22_v, ffn_sem.at[3]).wait()

    o1 = _ffn_block(m1, w11_v[...], l11b[...], w12_v[...], l12b[...],
                    n13g[...], n13b[...])
    o2 = _ffn_block(m2, w21_v[...], l21b[...], w22_v[...], l22b[...],
                    n23g[...], n23b[...])
    out1_ref[...] = o1.reshape(s, bt, d)
    out2_ref[...] = o2.reshape(s, bt, d)
    attn1_ref[...] = at1
    attn2_ref[...] = at2


def _vec(b):
    return b.reshape(1, b.shape[0])


def kernel(src1, src2, self_attn1_wq, self_attn1_wk, self_attn1_wv, self_attn1_wo, self_attn1_bq, self_attn1_bk, self_attn1_bv, self_attn1_bo, self_attn2_wq, self_attn2_wk, self_attn2_wv, self_attn2_wo, self_attn2_bq, self_attn2_bk, self_attn2_bv, self_attn2_bo, multihead_attn1_wq, multihead_attn1_wk, multihead_attn1_wv, multihead_attn1_wo, multihead_attn1_bq, multihead_attn1_bk, multihead_attn1_bv, multihead_attn1_bo, multihead_attn2_wq, multihead_attn2_wk, multihead_attn2_wv, multihead_attn2_wo, multihead_attn2_bq, multihead_attn2_bk, multihead_attn2_bv, multihead_attn2_bo, lin11_w, lin11_b, lin12_w, lin12_b, lin21_w, lin21_b, lin22_w, lin22_b, norm11_g, norm11_b, norm12_g, norm12_b, norm13_g, norm13_b, norm21_g, norm21_b, norm22_g, norm22_b, norm23_g, norm23_b):
    s, b, d = src1.shape
    hd = d // _NHEAD
    scale = 1.0 / (hd ** 0.5)
    bt = 8 if b % 8 == 0 else b
    nj = b // bt

    xspec = pl.BlockSpec((s, bt, d), lambda j: (0, j, 0))

    def wspec(arr):
        nd = arr.ndim
        return pl.BlockSpec(arr.shape, lambda j, _n=nd: (0,) * _n)

    ops1 = [
        src1, src2,
        self_attn1_wq, _vec(self_attn1_bq), self_attn1_wk, _vec(self_attn1_bk),
        self_attn1_wv, _vec(self_attn1_bv), self_attn1_wo, _vec(self_attn1_bo),
        _vec(norm11_g), _vec(norm11_b),
        self_attn2_wq, _vec(self_attn2_bq), self_attn2_wk, _vec(self_attn2_bk),
        self_attn2_wv, _vec(self_attn2_bv), self_attn2_wo, _vec(self_attn2_bo),
        _vec(norm21_g), _vec(norm21_b),
    ]
    y1, y2 = pl.pallas_call(
        functools.partial(_stage1_kernel, scale=scale),
        out_shape=(jax.ShapeDtypeStruct((s, b, d), jnp.float32),
                   jax.ShapeDtypeStruct((s, b, d), jnp.float32)),
        grid=(nj,),
        in_specs=[xspec, xspec] + [wspec(a) for a in ops1[2:]],
        out_specs=(xspec, xspec),
        compiler_params=pltpu.CompilerParams(
            dimension_semantics=("arbitrary",)),
    )(*ops1)

    f = lin11_w.shape[0]
    ops2 = [
        y1, y2,
        multihead_attn1_wq, _vec(multihead_attn1_bq),
        multihead_attn1_wk, _vec(multihead_attn1_bk),
        multihead_attn1_wv, _vec(multihead_attn1_bv),
        multihead_attn1_wo, _vec(multihead_attn1_bo),
        _vec(norm12_g), _vec(norm12_b),
        lin11_w, _vec(lin11_b), lin12_w, _vec(lin12_b),
        _vec(norm13_g), _vec(norm13_b),
        multihead_attn2_wq, _vec(multihead_attn2_bq),
        multihead_attn2_wk, _vec(multihead_attn2_bk),
        multihead_attn2_wv, _vec(multihead_attn2_bv),
        multihead_attn2_wo, _vec(multihead_attn2_bo),
        _vec(norm22_g), _vec(norm22_b),
        lin21_w, _vec(lin21_b), lin22_w, _vec(lin22_b),
        _vec(norm23_g), _vec(norm23_b),
    ]
    ffn_names = {12, 14, 28, 30}
    in_specs2 = []
    for i, a in enumerate(ops2):
        if i < 2:
            in_specs2.append(xspec)
        elif i in ffn_names:
            in_specs2.append(pl.BlockSpec(memory_space=pl.ANY))
        else:
            in_specs2.append(wspec(a))
    out1, out2, attn1, attn2 = pl.pallas_call(
        functools.partial(_stage2_kernel, scale=scale),
        out_shape=(jax.ShapeDtypeStruct((s, b, d), jnp.float32),
                   jax.ShapeDtypeStruct((s, b, d), jnp.float32),
                   jax.ShapeDtypeStruct((b, s, s), jnp.float32),
                   jax.ShapeDtypeStruct((b, s, s), jnp.float32)),
        grid=(nj,),
        in_specs=in_specs2,
        out_specs=(xspec, xspec,
                   pl.BlockSpec((bt, s, s), lambda j: (j, 0, 0)),
                   pl.BlockSpec((bt, s, s), lambda j: (j, 0, 0))),
        scratch_shapes=[pltpu.VMEM((f, d), jnp.float32),
                        pltpu.VMEM((d, f), jnp.float32),
                        pltpu.VMEM((f, d), jnp.float32),
                        pltpu.VMEM((d, f), jnp.float32),
                        pltpu.SemaphoreType.DMA((4,))],
        compiler_params=pltpu.CompilerParams(
            dimension_semantics=("arbitrary",)),
    )(*ops2)
    return out1, out2, attn1, attn2
